# baseline (speedup 1.0000x reference)
import functools

import jax
import jax.numpy as jnp
from jax import lax
from jax.experimental import pallas as pl
from jax.experimental.pallas import tpu as pltpu
from jax.experimental.pallas import tpu_sc as plsc

N = 10000
D = 128
DH = D // 2
E = 320000
EH = E // 2
RA = 475
RAP = 480
NP = 10000
NPD = 10112
NC = 2
NS = 16
ROWS_PER_TILE = NP // NS
DEG_PER_TILE = 2 * NPD // NS
CH = 128
NCH = EH // CH
NCH_ALL = E // CH
BLK = 400
GRID = N // BLK


def _tc1_node_body(x_ref, wi_ref, wo_ref, wl_ref, lr_ref, xw_ref, loop_ref):
    xb = x_ref[...]
    xwi = jnp.dot(xb, wi_ref[...], preferred_element_type=jnp.float32)
    xwo = jnp.dot(xb, wo_ref[...], preferred_element_type=jnp.float32)
    for h in range(2):
        xw_ref[h, 0] = xwi[:, h * DH:(h + 1) * DH]
        xw_ref[h, 1] = xwo[:, h * DH:(h + 1) * DH]
    loop_ref[...] = jnp.dot(xb - lr_ref[...], wl_ref[...],
                            preferred_element_type=jnp.float32)


def _tc1_rel_body(ra_ref, wi_ref, wo_ref, wr_ref, rw_ref, relout_ref):
    ra = ra_ref[...]
    rwi = jnp.dot(ra, wi_ref[...], preferred_element_type=jnp.float32)
    rwo = jnp.dot(ra, wo_ref[...], preferred_element_type=jnp.float32)
    for h in range(2):
        rw_ref[h, 0] = rwi[:, h * DH:(h + 1) * DH]
        rw_ref[h, 1] = rwo[:, h * DH:(h + 1) * DH]
    relout_ref[...] = jnp.dot(ra, wr_ref[...],
                              preferred_element_type=jnp.float32)


def _rsqrt16(x):
    xi = lax.bitcast_convert_type(x, jnp.int32)
    yi = jnp.int32(0x5F3759DF) - lax.shift_right_logical(xi, 1)
    y = lax.bitcast_convert_type(yi, jnp.float32)
    xh = x * 0.5
    for _ in range(3):
        y = y * (1.5 - xh * y * y)
    return y


def _zero_acc(rows_v, acc_s, t):
    for b in range(5):
        pltpu.sync_copy(rows_v.at[pl.ds(0, 125)],
                        acc_s.at[pl.ds(t * ROWS_PER_TILE + b * 125, 125)])


def _zero_rows(rows_v):
    z16 = jnp.zeros((16,), jnp.float32)

    def _zr(i, _):
        for j in range(DH // 16):
            rows_v[i, pl.ds(j * 16, 16)] = z16
        return 0
    lax.fori_loop(0, CH, _zr, 0)


def _sc_body(ei_ref, et_ref, xw_ref, rw_ref, acc_out_ref,
             rows0, rows1, rel0, rel1, dbuf_v, src0, src1, dst0, dst1,
             ety0, ety1, csi0, csi1, cdi0, cdi1, cs0, cs1, cd0, cd1,
             ones_v, acc_s, dinv_s, rw_s, *sems):
    c = lax.axis_index("c")
    t = lax.axis_index("s")
    rows = [rows0, rows1]
    relbuf = [rel0, rel1]
    src = [src0, src1]
    dst = [dst0, dst1]
    ety = [ety0, ety1]
    csi = [csi0, csi1]
    cdi = [cdi0, cdi1]
    cs = [cs0, cs1]
    cd = [cd0, cd1]
    rows_v = rows0

    z16 = jnp.zeros((16,), jnp.float32)
    _zero_rows(rows_v)

    def _zero_small(i, _):
        dbuf_v[pl.ds(i * 16, 16)] = z16
        return 0
    lax.fori_loop(0, DEG_PER_TILE // 16, _zero_small, 0)

    for j in range(CH // 16):
        ones_v[pl.ds(j * 16, 16)] = jnp.ones((16,), jnp.float32)

    @pl.when(t == 0)
    def _():
        pltpu.sync_copy(rw_ref.at[c], rw_s)

    _zero_acc(rows_v, acc_s, t)
    pltpu.sync_copy(dbuf_v, dinv_s.at[pl.ds(t * DEG_PER_TILE, DEG_PER_TILE)])
    plsc.subcore_barrier()

    nk = jnp.where(t < (NCH_ALL % NS), NCH_ALL // NS + 1, NCH_ALL // NS)

    def _hadj(kc, sl):
        cid = t + kc * NS
        off_nd = (cid // NCH) * NPD
        for g in range(CH // 16):
            w = pl.ds(g * 16, 16)
            dst[sl][w] = dst[sl][w] + off_nd

    def _hpair(kk, _):
        ka = 2 * kk
        kb = ka + 1
        ca = pltpu.async_copy(ei_ref.at[0, pl.ds((t + ka * NS) * CH, CH)],
                              dst[0], sems[0])
        cb = pltpu.async_copy(ei_ref.at[0, pl.ds((t + kb * NS) * CH, CH)],
                              dst[1], sems[1])
        ca.wait()
        _hadj(ka, 0)
        sa = pltpu.async_copy(ones_v, dinv_s.at[dst[0]], sems[2], add=True)
        cb.wait()
        _hadj(kb, 1)
        sa.wait()
        pltpu.sync_copy(ones_v, dinv_s.at[dst[1]], add=True)
        return 0
    lax.fori_loop(0, nk // 2, _hpair, 0)

    @pl.when(nk % 2 == 1)
    def _():
        kc = nk - 1
        pltpu.sync_copy(ei_ref.at[0, pl.ds((t + kc * NS) * CH, CH)], dst[0])
        _hadj(kc, 0)
        pltpu.sync_copy(ones_v, dinv_s.at[dst[0]], add=True)
    plsc.subcore_barrier()

    sl_t = pl.ds(t * DEG_PER_TILE, DEG_PER_TILE)
    pltpu.sync_copy(dinv_s.at[sl_t], dbuf_v)

    def _dinv(g, _):
        xv = dbuf_v[pl.ds(g * 16, 16)]
        y = _rsqrt16(xv)
        dbuf_v[pl.ds(g * 16, 16)] = jnp.where(xv > 0.5, y, 0.0)
        return 0
    lax.fori_loop(0, DEG_PER_TILE // 16, _dinv, 0)
    pltpu.sync_copy(dbuf_v, dinv_s.at[sl_t])
    plsc.subcore_barrier()

    xw_base = c * (2 * N)

    def _eload(kc, sl, s0):
        base = (t + kc * NS) * CH
        c1 = pltpu.async_copy(ei_ref.at[0, pl.ds(base, CH)], dst[sl],
                              sems[s0])
        c2 = pltpu.async_copy(ei_ref.at[1, pl.ds(base, CH)], src[sl],
                              sems[s0 + 1])
        c3 = pltpu.async_copy(et_ref.at[pl.ds(base, CH)], ety[sl],
                              sems[s0 + 2])
        return (c1, c2, c3)

    def _eadj(kc, sl):
        cid = t + kc * NS
        dirv = cid // NCH
        off_nd = dirv * NPD
        off_xw = xw_base + dirv * N
        off_rw = dirv * RAP
        for g in range(CH // 16):
            w = pl.ds(g * 16, 16)
            s16 = src[sl][w]
            d16 = dst[sl][w]
            csi[sl][w] = s16 + off_nd
            cdi[sl][w] = d16 + off_nd
            src[sl][w] = s16 + off_xw
            ety[sl][w] = ety[sl][w] + off_rw

    def _efire(sl, s0):
        g1 = pltpu.async_copy(xw_ref.at[src[sl]], rows[sl], sems[s0])
        g2 = pltpu.async_copy(rw_s.at[ety[sl]], relbuf[sl], sems[s0 + 1])
        g3 = pltpu.async_copy(dinv_s.at[csi[sl]], cs[sl], sems[s0 + 2])
        g4 = pltpu.async_copy(dinv_s.at[cdi[sl]], cd[sl], sems[s0 + 3])
        return (g1, g2, g3, g4)

    def _ecompose(sl):
        def _edge_group(g, _):
            c16 = cs[sl][pl.ds(g * 16, 16)] * cd[sl][pl.ds(g * 16, 16)]
            for e0 in range(16):
                e = g * 16 + e0
                c_e = c16[e0]
                for j in range(DH // 16):
                    w = pl.ds(j * 16, 16)
                    rows[sl][e, w] = c_e * (rows[sl][e, w]
                                            - relbuf[sl][e, w])
            return 0
        lax.fori_loop(0, CH // 16, _edge_group, 0)

    def _epair(kk, _):
        ka = 2 * kk
        kb = ka + 1
        la = _eload(ka, 0, 0)
        lb = _eload(kb, 1, 3)
        for cp in la:
            cp.wait()
        _eadj(ka, 0)
        ga = _efire(0, 6)
        for cp in lb:
            cp.wait()
        _eadj(kb, 1)
        gb = _efire(1, 10)
        for cp in ga:
            cp.wait()
        _ecompose(0)
        sa = pltpu.async_copy(rows[0], acc_s.at[dst[0]], sems[14], add=True)
        for cp in gb:
            cp.wait()
        _ecompose(1)
        sa.wait()
        pltpu.sync_copy(rows[1], acc_s.at[dst[1]], add=True)
        return 0
    lax.fori_loop(0, nk // 2, _epair, 0)

    @pl.when(nk % 2 == 1)
    def _():
        kc = nk - 1
        for cp in _eload(kc, 0, 0):
            cp.wait()
        _eadj(kc, 0)
        for cp in _efire(0, 6):
            cp.wait()
        _ecompose(0)
        pltpu.sync_copy(rows[0], acc_s.at[dst[0]], add=True)
    plsc.subcore_barrier()

    rbase = t * ROWS_PER_TILE
    pltpu.sync_copy(acc_s.at[pl.ds(rbase, ROWS_PER_TILE)],
                    acc_out_ref.at[c, pl.ds(rbase, ROWS_PER_TILE)])


_sc_edge_kernel = functools.partial(
    pl.kernel,
    out_type=jax.ShapeDtypeStruct((NC, NP, DH), jnp.float32),
    mesh=plsc.VectorSubcoreMesh(core_axis_name="c", subcore_axis_name="s",
                                num_cores=NC, num_subcores=NS),
    compiler_params=pltpu.CompilerParams(needs_layout_passes=False,
                                         use_tc_tiling_on_sc=False),
    scratch_types=(
        [pltpu.VMEM((CH, DH), jnp.float32)] * 4
        + [pltpu.VMEM((DEG_PER_TILE,), jnp.float32)]
        + [pltpu.VMEM((CH,), jnp.int32)] * 10
        + [pltpu.VMEM((CH,), jnp.float32)] * 4
        + [pltpu.VMEM((CH,), jnp.float32)]
        + [pltpu.VMEM_SHARED((NP, DH), jnp.float32)]
        + [pltpu.VMEM_SHARED((2 * NPD,), jnp.float32)]
        + [pltpu.VMEM_SHARED((2 * RAP, DH), jnp.float32)]
        + [pltpu.SemaphoreType.DMA] * 15
    ),
)(_sc_body)


def _tc2a_body(a0_ref, a1_ref, loop_ref, bias_ref, pre_ref, s1_ref, s2_ref):
    i = pl.program_id(0)

    @pl.when(i == 0)
    def _():
        s1_ref[...] = jnp.zeros_like(s1_ref)
        s2_ref[...] = jnp.zeros_like(s2_ref)

    msg = jnp.concatenate([a0_ref[0], a1_ref[0]], axis=1)
    pre = (msg + loop_ref[...]) * (1.0 / 3.0) + bias_ref[...]
    pre_ref[...] = pre
    s1_ref[...] += jnp.sum(pre, axis=0, keepdims=True)
    s2_ref[...] += jnp.sum(pre * pre, axis=0, keepdims=True)


def _tc2b_body(pre_ref, s1_ref, s2_ref, g_ref, b_ref, out_ref):
    mean = s1_ref[...] * (1.0 / N)
    var = s2_ref[...] * (1.0 / N) - mean * mean
    scale = lax.rsqrt(var + 1e-5) * g_ref[...]
    out_ref[...] = (pre_ref[...] - mean) * scale + b_ref[...]


def kernel(x, edge_index, edge_type, rel_embed, w_loop, w_in, w_out, w_rel,
           loop_rel, bias, bn_gamma, bn_beta):
    edge_index = edge_index.astype(jnp.int32)
    edge_type = edge_type.astype(jnp.int32)
    rel_all = jnp.concatenate([rel_embed, loop_rel], axis=0)
    ra_pad = jnp.concatenate(
        [rel_all, jnp.zeros((RAP - RA, D), jnp.float32)], axis=0)

    xw, loop_res = pl.pallas_call(
        _tc1_node_body,
        grid=(GRID,),
        in_specs=[
            pl.BlockSpec((BLK, D), lambda i: (i, 0)),
            pl.BlockSpec((D, D), lambda i: (0, 0)),
            pl.BlockSpec((D, D), lambda i: (0, 0)),
            pl.BlockSpec((D, D), lambda i: (0, 0)),
            pl.BlockSpec((1, D), lambda i: (0, 0)),
        ],
        out_specs=[
            pl.BlockSpec((NC, 2, BLK, DH), lambda i: (0, 0, i, 0)),
            pl.BlockSpec((BLK, D), lambda i: (i, 0)),
        ],
        out_shape=[
            jax.ShapeDtypeStruct((NC, 2, N, DH), jnp.float32),
            jax.ShapeDtypeStruct((N, D), jnp.float32),
        ],
    )(x, w_in, w_out, w_loop, loop_rel)

    rw, relout_full = pl.pallas_call(
        _tc1_rel_body,
        grid=(1,),
        in_specs=[
            pl.BlockSpec((RAP, D), lambda i: (0, 0)),
            pl.BlockSpec((D, D), lambda i: (0, 0)),
            pl.BlockSpec((D, D), lambda i: (0, 0)),
            pl.BlockSpec((D, D), lambda i: (0, 0)),
        ],
        out_specs=[
            pl.BlockSpec((NC, 2, RAP, DH), lambda i: (0, 0, 0, 0)),
            pl.BlockSpec((RAP, D), lambda i: (0, 0)),
        ],
        out_shape=[
            jax.ShapeDtypeStruct((NC, 2, RAP, DH), jnp.float32),
            jax.ShapeDtypeStruct((RAP, D), jnp.float32),
        ],
    )(ra_pad, w_in, w_out, w_rel)

    xw_flat = xw.reshape(NC * 2 * N, DH)
    rw_flat = rw.reshape(NC, 2 * RAP, DH)
    acc = _sc_edge_kernel(edge_index, edge_type, xw_flat, rw_flat)

    pre, s1, s2 = pl.pallas_call(
        _tc2a_body,
        grid=(GRID,),
        in_specs=[
            pl.BlockSpec((1, BLK, DH), lambda i: (0, i, 0)),
            pl.BlockSpec((1, BLK, DH), lambda i: (1, i, 0)),
            pl.BlockSpec((BLK, D), lambda i: (i, 0)),
            pl.BlockSpec((1, D), lambda i: (0, 0)),
        ],
        out_specs=[
            pl.BlockSpec((BLK, D), lambda i: (i, 0)),
            pl.BlockSpec((1, D), lambda i: (0, 0)),
            pl.BlockSpec((1, D), lambda i: (0, 0)),
        ],
        out_shape=[
            jax.ShapeDtypeStruct((N, D), jnp.float32),
            jax.ShapeDtypeStruct((1, D), jnp.float32),
            jax.ShapeDtypeStruct((1, D), jnp.float32),
        ],
    )(acc, acc, loop_res, bias.reshape(1, D))

    out = pl.pallas_call(
        _tc2b_body,
        grid=(GRID,),
        in_specs=[
            pl.BlockSpec((BLK, D), lambda i: (i, 0)),
            pl.BlockSpec((1, D), lambda i: (0, 0)),
            pl.BlockSpec((1, D), lambda i: (0, 0)),
            pl.BlockSpec((1, D), lambda i: (0, 0)),
            pl.BlockSpec((1, D), lambda i: (0, 0)),
        ],
        out_specs=pl.BlockSpec((BLK, D), lambda i: (i, 0)),
        out_shape=jax.ShapeDtypeStruct((N, D), jnp.float32),
    )(pre, s1, s2, bn_gamma.reshape(1, D), bn_beta.reshape(1, D))

    return (out, relout_full[:RA - 1])


# speedup vs baseline: 1.0336x; 1.0336x over previous
import functools

import jax
import jax.numpy as jnp
from jax import lax
from jax.experimental import pallas as pl
from jax.experimental.pallas import tpu as pltpu
from jax.experimental.pallas import tpu_sc as plsc

N = 10000
D = 128
DH = D // 2
E = 320000
EH = E // 2
RA = 475
RAP = 480
NP = 10000
NPD = 10112
NC = 2
NS = 16
ROWS_PER_TILE = NP // NS
DEG_PER_TILE = 2 * NPD // NS
CH = 128
NCH = EH // CH
NCH_ALL = E // CH
BLK = 400
GRID = N // BLK


def _tc1_node_body(x_ref, wi_ref, wo_ref, wl_ref, lr_ref, xw_ref, loop_ref):
    xb = x_ref[...]
    xwi = jnp.dot(xb, wi_ref[...], preferred_element_type=jnp.float32)
    xwo = jnp.dot(xb, wo_ref[...], preferred_element_type=jnp.float32)
    for h in range(2):
        xw_ref[h, 0] = xwi[:, h * DH:(h + 1) * DH]
        xw_ref[h, 1] = xwo[:, h * DH:(h + 1) * DH]
    loop_ref[...] = jnp.dot(xb - lr_ref[...], wl_ref[...],
                            preferred_element_type=jnp.float32)


def _tc1_rel_body(ra_ref, wi_ref, wo_ref, wr_ref, rw_ref, relout_ref):
    ra = ra_ref[...]
    rwi = jnp.dot(ra, wi_ref[...], preferred_element_type=jnp.float32)
    rwo = jnp.dot(ra, wo_ref[...], preferred_element_type=jnp.float32)
    for h in range(2):
        rw_ref[h, 0] = rwi[:, h * DH:(h + 1) * DH]
        rw_ref[h, 1] = rwo[:, h * DH:(h + 1) * DH]
    relout_ref[...] = jnp.dot(ra, wr_ref[...],
                              preferred_element_type=jnp.float32)


def _rsqrt16(x):
    xi = lax.bitcast_convert_type(x, jnp.int32)
    yi = jnp.int32(0x5F3759DF) - lax.shift_right_logical(xi, 1)
    y = lax.bitcast_convert_type(yi, jnp.float32)
    xh = x * 0.5
    for _ in range(3):
        y = y * (1.5 - xh * y * y)
    return y


def _zero_acc(rows_v, acc_s, t):
    for b in range(5):
        pltpu.sync_copy(rows_v.at[pl.ds(0, 125)],
                        acc_s.at[pl.ds(t * ROWS_PER_TILE + b * 125, 125)])


def _zero_rows(rows_v):
    z16 = jnp.zeros((16,), jnp.float32)

    def _zr(i, _):
        for j in range(DH // 16):
            rows_v[i, pl.ds(j * 16, 16)] = z16
        return 0
    lax.fori_loop(0, CH, _zr, 0)


def _sc_body(ei_ref, et_ref, xw_ref, rw_ref, acc_out_ref,
             rows0, rows1, rel0, rel1, dbuf_v, src0, src1, dst0, dst1,
             ety0, ety1, csi0, csi1, cdi0, cdi1, cs0, cs1, cd0, cd1,
             ones_v, acc_s, dinv_s, rw_s, *sems):
    c = lax.axis_index("c")
    t = lax.axis_index("s")
    rows = [rows0, rows1]
    relbuf = [rel0, rel1]
    src = [src0, src1]
    dst = [dst0, dst1]
    ety = [ety0, ety1]
    csi = [csi0, csi1]
    cdi = [cdi0, cdi1]
    cs = [cs0, cs1]
    cd = [cd0, cd1]
    rows_v = rows0

    z16 = jnp.zeros((16,), jnp.float32)
    _zero_rows(rows_v)

    def _zero_small(i, _):
        dbuf_v[pl.ds(i * 16, 16)] = z16
        return 0
    lax.fori_loop(0, DEG_PER_TILE // 16, _zero_small, 0)

    for j in range(CH // 16):
        ones_v[pl.ds(j * 16, 16)] = jnp.ones((16,), jnp.float32)

    @pl.when(t == 0)
    def _():
        pltpu.sync_copy(rw_ref.at[c], rw_s)

    _zero_acc(rows_v, acc_s, t)
    pltpu.sync_copy(dbuf_v, dinv_s.at[pl.ds(t * DEG_PER_TILE, DEG_PER_TILE)])
    plsc.subcore_barrier()

    nk = jnp.where(t < (NCH_ALL % NS), NCH_ALL // NS + 1, NCH_ALL // NS)

    def _hadj(kc, sl):
        cid = t + kc * NS
        off_nd = (cid // NCH) * NPD
        for g in range(CH // 16):
            w = pl.ds(g * 16, 16)
            dst[sl][w] = dst[sl][w] + off_nd

    def _hpair(kk, _):
        ka = 2 * kk
        kb = ka + 1
        ca = pltpu.async_copy(ei_ref.at[0, pl.ds((t + ka * NS) * CH, CH)],
                              dst[0], sems[0])
        cb = pltpu.async_copy(ei_ref.at[0, pl.ds((t + kb * NS) * CH, CH)],
                              dst[1], sems[1])
        ca.wait()
        _hadj(ka, 0)
        sa = pltpu.async_copy(ones_v, dinv_s.at[dst[0]], sems[2], add=True)
        cb.wait()
        _hadj(kb, 1)
        sa.wait()
        pltpu.sync_copy(ones_v, dinv_s.at[dst[1]], add=True)
        return 0
    lax.fori_loop(0, nk // 2, _hpair, 0)

    @pl.when(nk % 2 == 1)
    def _():
        kc = nk - 1
        pltpu.sync_copy(ei_ref.at[0, pl.ds((t + kc * NS) * CH, CH)], dst[0])
        _hadj(kc, 0)
        pltpu.sync_copy(ones_v, dinv_s.at[dst[0]], add=True)
    plsc.subcore_barrier()

    sl_t = pl.ds(t * DEG_PER_TILE, DEG_PER_TILE)
    pltpu.sync_copy(dinv_s.at[sl_t], dbuf_v)

    def _dinv(g, _):
        xv = dbuf_v[pl.ds(g * 16, 16)]
        y = _rsqrt16(xv)
        dbuf_v[pl.ds(g * 16, 16)] = jnp.where(xv > 0.5, y, 0.0)
        return 0
    lax.fori_loop(0, DEG_PER_TILE // 16, _dinv, 0)
    pltpu.sync_copy(dbuf_v, dinv_s.at[sl_t])
    plsc.subcore_barrier()

    xw_base = c * (2 * N)

    def _eload(kc, sl, s0):
        base = (t + kc * NS) * CH
        c1 = pltpu.async_copy(ei_ref.at[0, pl.ds(base, CH)], dst[sl],
                              sems[s0])
        c2 = pltpu.async_copy(ei_ref.at[1, pl.ds(base, CH)], src[sl],
                              sems[s0 + 1])
        c3 = pltpu.async_copy(et_ref.at[pl.ds(base, CH)], ety[sl],
                              sems[s0 + 2])
        return (c1, c2, c3)

    def _eadj(kc, sl):
        cid = t + kc * NS
        dirv = cid // NCH
        off_nd = dirv * NPD
        off_xw = xw_base + dirv * N
        off_rw = dirv * RAP
        for g in range(CH // 16):
            w = pl.ds(g * 16, 16)
            s16 = src[sl][w]
            d16 = dst[sl][w]
            csi[sl][w] = s16 + off_nd
            cdi[sl][w] = d16 + off_nd
            src[sl][w] = s16 + off_xw
            ety[sl][w] = ety[sl][w] + off_rw

    def _efire(sl, s0):
        g1 = pltpu.async_copy(xw_ref.at[src[sl]], rows[sl], sems[s0])
        g2 = pltpu.async_copy(rw_s.at[ety[sl]], relbuf[sl], sems[s0 + 1])
        g3 = pltpu.async_copy(dinv_s.at[csi[sl]], cs[sl], sems[s0 + 2])
        g4 = pltpu.async_copy(dinv_s.at[cdi[sl]], cd[sl], sems[s0 + 3])
        return (g1, g2, g3, g4)

    def _ecompose(sl):
        def _edge_group(g, _):
            c16 = cs[sl][pl.ds(g * 16, 16)] * cd[sl][pl.ds(g * 16, 16)]
            for e0 in range(16):
                e = g * 16 + e0
                c_e = c16[e0]
                for j in range(DH // 16):
                    w = pl.ds(j * 16, 16)
                    rows[sl][e, w] = c_e * (rows[sl][e, w]
                                            - relbuf[sl][e, w])
            return 0
        lax.fori_loop(0, CH // 16, _edge_group, 0)

    def _epair(kk, _):
        ka = 2 * kk
        kb = ka + 1
        la = _eload(ka, 0, 0)
        lb = _eload(kb, 1, 3)
        for cp in la:
            cp.wait()
        _eadj(ka, 0)
        ga = _efire(0, 6)
        for cp in lb:
            cp.wait()
        _eadj(kb, 1)
        for cp in ga:
            cp.wait()
        gb = _efire(1, 10)
        _ecompose(0)
        sa = pltpu.async_copy(rows[0], acc_s.at[dst[0]], sems[14], add=True)
        for cp in gb:
            cp.wait()
        _ecompose(1)
        sa.wait()
        pltpu.sync_copy(rows[1], acc_s.at[dst[1]], add=True)
        return 0
    lax.fori_loop(0, nk // 2, _epair, 0)

    @pl.when(nk % 2 == 1)
    def _():
        kc = nk - 1
        for cp in _eload(kc, 0, 0):
            cp.wait()
        _eadj(kc, 0)
        for cp in _efire(0, 6):
            cp.wait()
        _ecompose(0)
        pltpu.sync_copy(rows[0], acc_s.at[dst[0]], add=True)
    plsc.subcore_barrier()

    rbase = t * ROWS_PER_TILE
    pltpu.sync_copy(acc_s.at[pl.ds(rbase, ROWS_PER_TILE)],
                    acc_out_ref.at[c, pl.ds(rbase, ROWS_PER_TILE)])


_sc_edge_kernel = functools.partial(
    pl.kernel,
    out_type=jax.ShapeDtypeStruct((NC, NP, DH), jnp.float32),
    mesh=plsc.VectorSubcoreMesh(core_axis_name="c", subcore_axis_name="s",
                                num_cores=NC, num_subcores=NS),
    compiler_params=pltpu.CompilerParams(needs_layout_passes=False,
                                         use_tc_tiling_on_sc=False),
    scratch_types=(
        [pltpu.VMEM((CH, DH), jnp.float32)] * 4
        + [pltpu.VMEM((DEG_PER_TILE,), jnp.float32)]
        + [pltpu.VMEM((CH,), jnp.int32)] * 10
        + [pltpu.VMEM((CH,), jnp.float32)] * 4
        + [pltpu.VMEM((CH,), jnp.float32)]
        + [pltpu.VMEM_SHARED((NP, DH), jnp.float32)]
        + [pltpu.VMEM_SHARED((2 * NPD,), jnp.float32)]
        + [pltpu.VMEM_SHARED((2 * RAP, DH), jnp.float32)]
        + [pltpu.SemaphoreType.DMA] * 15
    ),
)(_sc_body)


def _tc2a_body(a0_ref, a1_ref, loop_ref, bias_ref, pre_ref, s1_ref, s2_ref):
    i = pl.program_id(0)

    @pl.when(i == 0)
    def _():
        s1_ref[...] = jnp.zeros_like(s1_ref)
        s2_ref[...] = jnp.zeros_like(s2_ref)

    msg = jnp.concatenate([a0_ref[0], a1_ref[0]], axis=1)
    pre = (msg + loop_ref[...]) * (1.0 / 3.0) + bias_ref[...]
    pre_ref[...] = pre
    s1_ref[...] += jnp.sum(pre, axis=0, keepdims=True)
    s2_ref[...] += jnp.sum(pre * pre, axis=0, keepdims=True)


def _tc2b_body(pre_ref, s1_ref, s2_ref, g_ref, b_ref, out_ref):
    mean = s1_ref[...] * (1.0 / N)
    var = s2_ref[...] * (1.0 / N) - mean * mean
    scale = lax.rsqrt(var + 1e-5) * g_ref[...]
    out_ref[...] = (pre_ref[...] - mean) * scale + b_ref[...]


def kernel(x, edge_index, edge_type, rel_embed, w_loop, w_in, w_out, w_rel,
           loop_rel, bias, bn_gamma, bn_beta):
    edge_index = edge_index.astype(jnp.int32)
    edge_type = edge_type.astype(jnp.int32)
    rel_all = jnp.concatenate([rel_embed, loop_rel], axis=0)
    ra_pad = jnp.concatenate(
        [rel_all, jnp.zeros((RAP - RA, D), jnp.float32)], axis=0)

    xw, loop_res = pl.pallas_call(
        _tc1_node_body,
        grid=(GRID,),
        in_specs=[
            pl.BlockSpec((BLK, D), lambda i: (i, 0)),
            pl.BlockSpec((D, D), lambda i: (0, 0)),
            pl.BlockSpec((D, D), lambda i: (0, 0)),
            pl.BlockSpec((D, D), lambda i: (0, 0)),
            pl.BlockSpec((1, D), lambda i: (0, 0)),
        ],
        out_specs=[
            pl.BlockSpec((NC, 2, BLK, DH), lambda i: (0, 0, i, 0)),
            pl.BlockSpec((BLK, D), lambda i: (i, 0)),
        ],
        out_shape=[
            jax.ShapeDtypeStruct((NC, 2, N, DH), jnp.float32),
            jax.ShapeDtypeStruct((N, D), jnp.float32),
        ],
    )(x, w_in, w_out, w_loop, loop_rel)

    rw, relout_full = pl.pallas_call(
        _tc1_rel_body,
        grid=(1,),
        in_specs=[
            pl.BlockSpec((RAP, D), lambda i: (0, 0)),
            pl.BlockSpec((D, D), lambda i: (0, 0)),
            pl.BlockSpec((D, D), lambda i: (0, 0)),
            pl.BlockSpec((D, D), lambda i: (0, 0)),
        ],
        out_specs=[
            pl.BlockSpec((NC, 2, RAP, DH), lambda i: (0, 0, 0, 0)),
            pl.BlockSpec((RAP, D), lambda i: (0, 0)),
        ],
        out_shape=[
            jax.ShapeDtypeStruct((NC, 2, RAP, DH), jnp.float32),
            jax.ShapeDtypeStruct((RAP, D), jnp.float32),
        ],
    )(ra_pad, w_in, w_out, w_rel)

    xw_flat = xw.reshape(NC * 2 * N, DH)
    rw_flat = rw.reshape(NC, 2 * RAP, DH)
    acc = _sc_edge_kernel(edge_index, edge_type, xw_flat, rw_flat)

    pre, s1, s2 = pl.pallas_call(
        _tc2a_body,
        grid=(GRID,),
        in_specs=[
            pl.BlockSpec((1, BLK, DH), lambda i: (0, i, 0)),
            pl.BlockSpec((1, BLK, DH), lambda i: (1, i, 0)),
            pl.BlockSpec((BLK, D), lambda i: (i, 0)),
            pl.BlockSpec((1, D), lambda i: (0, 0)),
        ],
        out_specs=[
            pl.BlockSpec((BLK, D), lambda i: (i, 0)),
            pl.BlockSpec((1, D), lambda i: (0, 0)),
            pl.BlockSpec((1, D), lambda i: (0, 0)),
        ],
        out_shape=[
            jax.ShapeDtypeStruct((N, D), jnp.float32),
            jax.ShapeDtypeStruct((1, D), jnp.float32),
            jax.ShapeDtypeStruct((1, D), jnp.float32),
        ],
    )(acc, acc, loop_res, bias.reshape(1, D))

    out = pl.pallas_call(
        _tc2b_body,
        grid=(GRID,),
        in_specs=[
            pl.BlockSpec((BLK, D), lambda i: (i, 0)),
            pl.BlockSpec((1, D), lambda i: (0, 0)),
            pl.BlockSpec((1, D), lambda i: (0, 0)),
            pl.BlockSpec((1, D), lambda i: (0, 0)),
            pl.BlockSpec((1, D), lambda i: (0, 0)),
        ],
        out_specs=pl.BlockSpec((BLK, D), lambda i: (i, 0)),
        out_shape=jax.ShapeDtypeStruct((N, D), jnp.float32),
    )(pre, s1, s2, bn_gamma.reshape(1, D), bn_beta.reshape(1, D))

    return (out, relout_full[:RA - 1])
